# baseline (speedup 1.0000x reference)
import functools

import jax
import jax.numpy as jnp
from jax import lax
from jax.experimental import pallas as pl
from jax.experimental.pallas import tpu as pltpu
from jax.experimental.pallas import tpu_sc as plsc

N = 10000
E = 320000
F_IN = 128
H = 128
OUT = 64

NPAD = 10240
NC = 2
NS = 16
NW = NC * NS
EPT = E // NW
CHUNK = 128
NCH = 80
EPAD = NW * NCH * CHUNK
RPT = NPAD // NS
ZR = 64

_mesh = plsc.VectorSubcoreMesh(core_axis_name="c", subcore_axis_name="s")
_sc_params = pltpu.CompilerParams(needs_layout_passes=False)


def _deg_body(col_hbm, out_hbm, colbuf, hist):
    cid = lax.axis_index("c")
    sid = lax.axis_index("s")
    wid = sid * NC + cid
    pltpu.sync_copy(col_hbm.at[wid], colbuf)

    def zero(j, carry):
        hist[pl.ds(j * 16, 16)] = jnp.zeros((16,), jnp.float32)
        return carry

    lax.fori_loop(0, NPAD // 16, zero, 0)

    ones = jnp.ones((16,), jnp.float32)

    def body(j, carry):
        idx = colbuf[pl.ds(j * 16, 16)]
        plsc.addupdate_scatter(hist, [idx], ones)
        return carry

    lax.fori_loop(0, EPT // 16, body, 0)
    pltpu.sync_copy(hist, out_hbm.at[wid])


_deg = pl.kernel(
    _deg_body,
    mesh=_mesh,
    out_type=jax.ShapeDtypeStruct((NW, NPAD), jnp.float32),
    scratch_types=[
        pltpu.VMEM((EPT,), jnp.int32),
        pltpu.VMEM((NPAD,), jnp.float32),
    ],
    compiler_params=_sc_params,
)


def _scatter_body(feat, g_hbm, rc_hbm, out_hbm,
                  ibuf, gbuf, zbuf, s_sh,
                  gsem0, gsem1, isem0, isem1, isem2, isem3):
    cid = lax.axis_index("c")
    sid = lax.axis_index("s")
    wid = sid * NC + cid
    base = wid * NCH
    gsems = (gsem0, gsem1)
    isems = (isem0, isem1, isem2, isem3)

    for q in range(4):
        pltpu.async_copy(rc_hbm.at[base + q], ibuf.at[q], isems[q])
    for u in range(2):
        pltpu.make_async_copy(rc_hbm.at[base + u], ibuf.at[u],
                              isems[u]).wait()
        pltpu.async_copy(g_hbm.at[ibuf.at[u, 0]], gbuf.at[u], gsems[u])

    def zrow(r, carry):
        def zcol(c, carry2):
            zbuf[r, pl.ds(c * 16, 16)] = jnp.zeros((16,), jnp.float32)
            return carry2

        return lax.fori_loop(0, feat // 16, zcol, carry)

    lax.fori_loop(0, ZR, zrow, 0)

    def zfill(r, carry):
        pltpu.sync_copy(zbuf, s_sh.at[pl.ds(sid * RPT + r * ZR, ZR)])
        return carry

    lax.fori_loop(0, RPT // ZR, zfill, 0)
    plsc.subcore_barrier()

    def quad(p, carry):
        j0 = 4 * p
        for u in range(4):
            j = j0 + u
            b = u % 2
            pltpu.make_async_copy(g_hbm.at[ibuf.at[u, 0]], gbuf.at[b],
                                  gsems[b]).wait()
            pltpu.sync_copy(gbuf.at[b], s_sh.at[ibuf.at[u, 1]], add=True)

            @pl.when(j + 2 < NCH)
            def _():
                q2 = (u + 2) % 4
                pltpu.make_async_copy(rc_hbm.at[base + j + 2], ibuf.at[q2],
                                      isems[q2]).wait()
                pltpu.async_copy(g_hbm.at[ibuf.at[q2, 0]], gbuf.at[b],
                                 gsems[b])

            @pl.when(j + 4 < NCH)
            def _():
                pltpu.async_copy(rc_hbm.at[base + j + 4], ibuf.at[u],
                                 isems[u])

        return carry

    lax.fori_loop(0, NCH // 4, quad, 0)
    plsc.subcore_barrier()

    def out(r, carry):
        sl = pl.ds(sid * RPT + r * ZR, ZR)
        pltpu.sync_copy(s_sh.at[sl], out_hbm.at[cid, sl])
        return carry

    lax.fori_loop(0, RPT // ZR, out, 0)


def _make_scatter(feat):
    return pl.kernel(
        functools.partial(_scatter_body, feat),
        mesh=_mesh,
        out_type=jax.ShapeDtypeStruct((NC, NPAD, feat), jnp.float32),
        scratch_types=[
            pltpu.VMEM((4, 2, CHUNK), jnp.int32),
            pltpu.VMEM((2, CHUNK, feat), jnp.float32),
            pltpu.VMEM((ZR, feat), jnp.float32),
            pltpu.VMEM_SHARED((NPAD, feat), jnp.float32),
        ] + [pltpu.SemaphoreType.DMA] * 6,
        compiler_params=_sc_params,
    )


_scatter_h = _make_scatter(H)


NB = 1024


def _dinv_of(hist_ref):
    deg = jnp.sum(hist_ref[...], axis=0) + 1.0
    return lax.rsqrt(deg)[:, None]


def _tc1_body(hist_ref, x_ref, w_ref, g_ref):
    h = jnp.dot(x_ref[...], w_ref[...], preferred_element_type=jnp.float32)
    g_ref[...] = h * _dinv_of(hist_ref)


def _tc2_body(hist_ref, s_ref, g1_ref, b1_ref, w2_ref, g2_ref):
    dinv = _dinv_of(hist_ref)
    s = s_ref[0] + s_ref[1] + g1_ref[...]
    h1 = jnp.maximum(s * dinv + b1_ref[...], 0.0)
    g2_ref[...] = jnp.dot(h1, w2_ref[...],
                          preferred_element_type=jnp.float32) * dinv


def _tc3_body(hist_ref, s_ref, g2_ref, b2_ref, z_ref):
    s = s_ref[0] + s_ref[1] + g2_ref[...]
    z_ref[...] = s * _dinv_of(hist_ref) + b2_ref[...]


_GRID = (NPAD // NB,)
_hist_spec = pl.BlockSpec((NW, NB), lambda i: (0, i))


def _row_spec(f):
    return pl.BlockSpec((NB, f), lambda i: (i, 0))


def _part_spec(f):
    return pl.BlockSpec((NC, NB, f), lambda i: (0, i, 0))


def _full_spec(r, c):
    return pl.BlockSpec((r, c), lambda i: (0, 0))


_tc1 = pl.pallas_call(
    _tc1_body,
    grid=_GRID,
    in_specs=[_hist_spec, _row_spec(F_IN), _full_spec(F_IN, H)],
    out_specs=_row_spec(H),
    out_shape=jax.ShapeDtypeStruct((NPAD, H), jnp.float32),
)

_tc2 = pl.pallas_call(
    _tc2_body,
    grid=_GRID,
    in_specs=[_hist_spec, _part_spec(H), _row_spec(H),
              _full_spec(1, H), _full_spec(H, H)],
    out_specs=_row_spec(H),
    out_shape=jax.ShapeDtypeStruct((NPAD, H), jnp.float32),
)

_tc3 = pl.pallas_call(
    _tc3_body,
    grid=_GRID,
    in_specs=[_hist_spec, _part_spec(H), _row_spec(H), _full_spec(1, H)],
    out_specs=_row_spec(H),
    out_shape=jax.ShapeDtypeStruct((NPAD, H), jnp.float32),
)


def kernel(x, edge_index, W1, b1, W2, b2):
    ei = edge_index.astype(jnp.int32)
    col_t = ei[1].reshape(NW, EPT)
    eip = jnp.pad(ei, ((0, 0), (0, EPAD - E)), constant_values=N)
    rc = eip.reshape(2, NW, NCH, CHUNK).transpose(1, 2, 0, 3).reshape(
        NW * NCH, 2, CHUNK)
    xp = jnp.pad(x, ((0, NPAD - N), (0, 0)))

    w2p = jnp.pad(W2, ((0, 0), (0, H - OUT)))
    b2p = jnp.pad(b2, (0, H - OUT)).reshape(1, H)

    hist = _deg(col_t)
    g1 = _tc1(hist, xp, W1)
    s1 = _scatter_h(g1, rc)
    g2 = _tc2(hist, s1, g1, b1.reshape(1, H), w2p)
    s2 = _scatter_h(g2, rc)
    z = _tc3(hist, s2, g2, b2p)
    return z[:N, :OUT]


# speedup vs baseline: 3.2008x; 3.2008x over previous
import functools

import jax
import jax.numpy as jnp
from jax import lax
from jax.experimental import pallas as pl
from jax.experimental.pallas import tpu as pltpu
from jax.experimental.pallas import tpu_sc as plsc

N = 10000
E = 320000
F_IN = 128
H = 128
OUT = 64

NPAD = 10240
NC = 2
NS = 16
NW = NC * NS
EPT = E // NW
CHUNK = 128
NCH = 80
EPAD = NW * NCH * CHUNK
RPT = NPAD // NS
ZR = 64

_mesh = plsc.VectorSubcoreMesh(core_axis_name="c", subcore_axis_name="s")
_sc_params = pltpu.CompilerParams(needs_layout_passes=False)


def _deg_body(col_hbm, out_hbm, colbuf, hist):
    cid = lax.axis_index("c")
    sid = lax.axis_index("s")
    wid = sid * NC + cid
    pltpu.sync_copy(col_hbm.at[wid], colbuf)

    def zero(j, carry):
        hist[pl.ds(j * 16, 16)] = jnp.zeros((16,), jnp.float32)
        return carry

    lax.fori_loop(0, NPAD // 16, zero, 0)

    ones = jnp.ones((16,), jnp.float32)

    def body(j, carry):
        idx = colbuf[pl.ds(j * 16, 16)]
        plsc.addupdate_scatter(hist, [idx], ones)
        return carry

    lax.fori_loop(0, EPT // 16, body, 0)
    pltpu.sync_copy(hist, out_hbm.at[wid])


_deg = pl.kernel(
    _deg_body,
    mesh=_mesh,
    out_type=jax.ShapeDtypeStruct((NW, NPAD), jnp.float32),
    scratch_types=[
        pltpu.VMEM((EPT,), jnp.int32),
        pltpu.VMEM((NPAD,), jnp.float32),
    ],
    compiler_params=_sc_params,
)


def _scatter_body(feat, g_hbm, rc_hbm, out_hbm,
                  ibuf, gbuf, zbuf, s_sh,
                  gsem0, gsem1, isem0, isem1, isem2, isem3):
    cid = lax.axis_index("c")
    sid = lax.axis_index("s")
    wid = sid * NC + cid
    base = wid * NCH
    gsems = (gsem0, gsem1)
    isems = (isem0, isem1, isem2, isem3)

    for q in range(4):
        pltpu.async_copy(rc_hbm.at[base + q], ibuf.at[q], isems[q])
    for u in range(2):
        pltpu.make_async_copy(rc_hbm.at[base + u], ibuf.at[u],
                              isems[u]).wait()
        pltpu.async_copy(g_hbm.at[ibuf.at[u, 0]], gbuf.at[u], gsems[u])

    def zrow(r, carry):
        def zcol(c, carry2):
            zbuf[r, pl.ds(c * 16, 16)] = jnp.zeros((16,), jnp.float32)
            return carry2

        return lax.fori_loop(0, feat // 16, zcol, carry)

    lax.fori_loop(0, ZR, zrow, 0)

    def zfill(r, carry):
        pltpu.sync_copy(zbuf, s_sh.at[pl.ds(sid * RPT + r * ZR, ZR)])
        return carry

    lax.fori_loop(0, RPT // ZR, zfill, 0)
    plsc.subcore_barrier()

    def quad(p, carry):
        j0 = 4 * p
        for u in range(4):
            j = j0 + u
            b = u % 2
            pltpu.make_async_copy(g_hbm.at[ibuf.at[u, 0]], gbuf.at[b],
                                  gsems[b]).wait()
            pltpu.sync_copy(gbuf.at[b], s_sh.at[ibuf.at[u, 1]], add=True)

            @pl.when(j + 2 < NCH)
            def _():
                q2 = (u + 2) % 4
                pltpu.make_async_copy(rc_hbm.at[base + j + 2], ibuf.at[q2],
                                      isems[q2]).wait()
                pltpu.async_copy(g_hbm.at[ibuf.at[q2, 0]], gbuf.at[b],
                                 gsems[b])

            @pl.when(j + 4 < NCH)
            def _():
                pltpu.async_copy(rc_hbm.at[base + j + 4], ibuf.at[u],
                                 isems[u])

        return carry

    lax.fori_loop(0, NCH // 4, quad, 0)
    plsc.subcore_barrier()

    def out(r, carry):
        sl = pl.ds(sid * RPT + r * ZR, ZR)
        pltpu.sync_copy(s_sh.at[sl], out_hbm.at[cid, sl])
        return carry

    lax.fori_loop(0, RPT // ZR, out, 0)


def _make_scatter(feat):
    return pl.kernel(
        functools.partial(_scatter_body, feat),
        mesh=_mesh,
        out_type=jax.ShapeDtypeStruct((NC, NPAD, feat), jnp.float32),
        scratch_types=[
            pltpu.VMEM((4, 2, CHUNK), jnp.int32),
            pltpu.VMEM((2, CHUNK, feat), jnp.float32),
            pltpu.VMEM((ZR, feat), jnp.float32),
            pltpu.VMEM_SHARED((NPAD, feat), jnp.float32),
        ] + [pltpu.SemaphoreType.DMA] * 6,
        compiler_params=_sc_params,
    )


_scatter_h = _make_scatter(H)


NB = 1024


def _dinv_of(hist_ref):
    deg = jnp.sum(hist_ref[...], axis=0) + 1.0
    return lax.rsqrt(deg)[:, None]


def _tc1_body(hist_ref, x_ref, w_ref, g_ref):
    h = jnp.dot(x_ref[...], w_ref[...], preferred_element_type=jnp.float32)
    g_ref[...] = h * _dinv_of(hist_ref)


def _tc2_body(hist_ref, s_ref, g1_ref, b1_ref, w2_ref, g2_ref):
    dinv = _dinv_of(hist_ref)
    s = s_ref[0] + s_ref[1] + g1_ref[...]
    h1 = jnp.maximum(s * dinv + b1_ref[...], 0.0)
    g2_ref[...] = jnp.dot(h1, w2_ref[...],
                          preferred_element_type=jnp.float32) * dinv


def _tc3_body(hist_ref, s_ref, g2_ref, b2_ref, z_ref):
    s = s_ref[0] + s_ref[1] + g2_ref[...]
    z_ref[...] = s * _dinv_of(hist_ref) + b2_ref[...]


_GRID = (NPAD // NB,)
_hist_spec = pl.BlockSpec((NW, NB), lambda i: (0, i))


def _row_spec(f):
    return pl.BlockSpec((NB, f), lambda i: (i, 0))


def _part_spec(f):
    return pl.BlockSpec((NC, NB, f), lambda i: (0, i, 0))


def _full_spec(r, c):
    return pl.BlockSpec((r, c), lambda i: (0, 0))


_tc1 = pl.pallas_call(
    _tc1_body,
    grid=_GRID,
    in_specs=[_hist_spec, _row_spec(F_IN), _full_spec(F_IN, H)],
    out_specs=_row_spec(H),
    out_shape=jax.ShapeDtypeStruct((NPAD, H), jnp.float32),
)

_tc2 = pl.pallas_call(
    _tc2_body,
    grid=_GRID,
    in_specs=[_hist_spec, _part_spec(H), _row_spec(H),
              _full_spec(1, H), _full_spec(H, H)],
    out_specs=_row_spec(H),
    out_shape=jax.ShapeDtypeStruct((NPAD, H), jnp.float32),
)

_tc3 = pl.pallas_call(
    _tc3_body,
    grid=_GRID,
    in_specs=[_hist_spec, _part_spec(H), _row_spec(H), _full_spec(1, H)],
    out_specs=_row_spec(H),
    out_shape=jax.ShapeDtypeStruct((NPAD, H), jnp.float32),
)


def kernel(x, edge_index, W1, b1, W2, b2):
    ei = edge_index.astype(jnp.int32)
    col_t = ei[1].reshape(NW, EPT)
    sent = N + jnp.arange(EPAD - E, dtype=jnp.int32) % (NPAD - N)
    eip = jnp.concatenate(
        [ei, jnp.broadcast_to(sent, (2, EPAD - E))], axis=1)
    rc = eip.reshape(2, NW, NCH, CHUNK).transpose(1, 2, 0, 3).reshape(
        NW * NCH, 2, CHUNK)
    xp = jnp.pad(x, ((0, NPAD - N), (0, 0)))

    w2p = jnp.pad(W2, ((0, 0), (0, H - OUT)))
    b2p = jnp.pad(b2, (0, H - OUT)).reshape(1, H)

    hist = _deg(col_t)
    g1 = _tc1(hist, xp, W1)
    s1 = _scatter_h(g1, rc)
    g2 = _tc2(hist, s1, g1, b1.reshape(1, H), w2p)
    s2 = _scatter_h(g2, rc)
    z = _tc3(hist, s2, g2, b2p)
    return z[:N, :OUT]


# speedup vs baseline: 3.2562x; 1.0173x over previous
import functools

import jax
import jax.numpy as jnp
from jax import lax
from jax.experimental import pallas as pl
from jax.experimental.pallas import tpu as pltpu
from jax.experimental.pallas import tpu_sc as plsc

N = 10000
E = 320000
F_IN = 128
H = 128
OUT = 64

NPAD = 10240
NC = 2
NS = 16
NW = NC * NS
EPT = E // NW
CHUNK = 120
NCH = 84
EPAD = NW * NCH * CHUNK
RPT = NPAD // NS
ZR = 64

_mesh = plsc.VectorSubcoreMesh(core_axis_name="c", subcore_axis_name="s")
_sc_params = pltpu.CompilerParams(needs_layout_passes=False)


def _deg_body(col_hbm, out_hbm, colbuf, hist):
    cid = lax.axis_index("c")
    sid = lax.axis_index("s")
    wid = sid * NC + cid
    pltpu.sync_copy(col_hbm.at[wid], colbuf)

    def zero(j, carry):
        hist[pl.ds(j * 16, 16)] = jnp.zeros((16,), jnp.float32)
        return carry

    lax.fori_loop(0, NPAD // 16, zero, 0)

    ones = jnp.ones((16,), jnp.float32)

    def body(j, carry):
        idx = colbuf[pl.ds(j * 16, 16)]
        plsc.addupdate_scatter(hist, [idx], ones)
        return carry

    lax.fori_loop(0, EPT // 16, body, 0)
    pltpu.sync_copy(hist, out_hbm.at[wid])


_deg = pl.kernel(
    _deg_body,
    mesh=_mesh,
    out_type=jax.ShapeDtypeStruct((NW, NPAD), jnp.float32),
    scratch_types=[
        pltpu.VMEM((EPT,), jnp.int32),
        pltpu.VMEM((NPAD,), jnp.float32),
    ],
    compiler_params=_sc_params,
)


def _scatter_body(feat, g_hbm, rc_hbm, out_hbm,
                  ibuf, gbuf, s_sh,
                  gsem0, gsem1, gsem2, ssem0, ssem1, ssem2,
                  isem0, isem1, isem2, isem3, isem4, isem5):
    cid = lax.axis_index("c")
    sid = lax.axis_index("s")
    wid = sid * NC + cid
    base = wid * NCH
    gsems = (gsem0, gsem1, gsem2)
    ssems = (ssem0, ssem1, ssem2)
    isems = (isem0, isem1, isem2, isem3, isem4, isem5)

    for q in range(4):
        pltpu.async_copy(rc_hbm.at[base + q], ibuf.at[q], isems[q])

    def zrow(r, carry):
        def zcol(c, carry2):
            gbuf[2, r, pl.ds(c * 16, 16)] = jnp.zeros((16,), jnp.float32)
            return carry2

        return lax.fori_loop(0, feat // 16, zcol, carry)

    lax.fori_loop(0, CHUNK, zrow, 0)

    def zfill(r, carry):
        pltpu.sync_copy(gbuf.at[2],
                        s_sh.at[pl.ds(sid * RPT + r * CHUNK, CHUNK)])
        return carry

    lax.fori_loop(0, RPT // CHUNK, zfill, 0)
    if RPT % CHUNK:
        pltpu.sync_copy(
            gbuf.at[2].at[pl.ds(0, RPT % CHUNK)],
            s_sh.at[pl.ds(sid * RPT + (RPT // CHUNK) * CHUNK, RPT % CHUNK)])

    for u in range(2):
        pltpu.make_async_copy(rc_hbm.at[base + u], ibuf.at[u],
                              isems[u]).wait()
        pltpu.async_copy(g_hbm.at[ibuf.at[u, 0]], gbuf.at[u], gsems[u])
    plsc.subcore_barrier()

    def six(p, carry):
        j0 = 6 * p
        for u in range(6):
            j = j0 + u
            b = u % 3
            b2 = (u + 2) % 3
            q2 = (u + 2) % 6
            q4 = (u + 4) % 6
            pltpu.make_async_copy(g_hbm.at[ibuf.at[u, 0]], gbuf.at[b],
                                  gsems[b]).wait()
            pltpu.async_copy(gbuf.at[b], s_sh.at[ibuf.at[u, 1]],
                             ssems[b], add=True)

            @pl.when(j + 2 < NCH)
            def _():
                @pl.when(j >= 1)
                def _():
                    pltpu.make_async_copy(gbuf.at[b2], s_sh.at[ibuf.at[u, 1]],
                                          ssems[b2]).wait()

                pltpu.make_async_copy(rc_hbm.at[base + j + 2], ibuf.at[q2],
                                      isems[q2]).wait()
                pltpu.async_copy(g_hbm.at[ibuf.at[q2, 0]], gbuf.at[b2],
                                 gsems[b2])

            @pl.when(j + 4 < NCH)
            def _():
                pltpu.async_copy(rc_hbm.at[base + j + 4], ibuf.at[q4],
                                 isems[q4])

        return carry

    lax.fori_loop(0, NCH // 6, six, 0)
    for j in (NCH - 3, NCH - 2, NCH - 1):
        b = j % 3
        pltpu.make_async_copy(gbuf.at[b], s_sh.at[ibuf.at[j % 6, 1]],
                              ssems[b]).wait()
    plsc.subcore_barrier()

    def out(r, carry):
        sl = pl.ds(sid * RPT + r * ZR, ZR)
        pltpu.sync_copy(s_sh.at[sl], out_hbm.at[cid, sl])
        return carry

    lax.fori_loop(0, RPT // ZR, out, 0)


def _make_scatter(feat):
    return pl.kernel(
        functools.partial(_scatter_body, feat),
        mesh=_mesh,
        out_type=jax.ShapeDtypeStruct((NC, NPAD, feat), jnp.float32),
        scratch_types=[
            pltpu.VMEM((6, 2, CHUNK), jnp.int32),
            pltpu.VMEM((3, CHUNK, feat), jnp.float32),
            pltpu.VMEM_SHARED((NPAD, feat), jnp.float32),
        ] + [pltpu.SemaphoreType.DMA] * 12,
        compiler_params=_sc_params,
    )


_scatter_h = _make_scatter(H)


NB = 1024


def _dinv_of(hist_ref):
    deg = jnp.sum(hist_ref[...], axis=0) + 1.0
    return lax.rsqrt(deg)[:, None]


def _tc1_body(hist_ref, x_ref, w_ref, g_ref):
    h = jnp.dot(x_ref[...], w_ref[...], preferred_element_type=jnp.float32)
    g_ref[...] = h * _dinv_of(hist_ref)


def _tc2_body(hist_ref, s_ref, g1_ref, b1_ref, w2_ref, g2_ref):
    dinv = _dinv_of(hist_ref)
    s = s_ref[0] + s_ref[1] + g1_ref[...]
    h1 = jnp.maximum(s * dinv + b1_ref[...], 0.0)
    g2_ref[...] = jnp.dot(h1, w2_ref[...],
                          preferred_element_type=jnp.float32) * dinv


def _tc3_body(hist_ref, s_ref, g2_ref, b2_ref, z_ref):
    s = s_ref[0] + s_ref[1] + g2_ref[...]
    z_ref[...] = s * _dinv_of(hist_ref) + b2_ref[...]


_GRID = (NPAD // NB,)
_hist_spec = pl.BlockSpec((NW, NB), lambda i: (0, i))


def _row_spec(f):
    return pl.BlockSpec((NB, f), lambda i: (i, 0))


def _part_spec(f):
    return pl.BlockSpec((NC, NB, f), lambda i: (0, i, 0))


def _full_spec(r, c):
    return pl.BlockSpec((r, c), lambda i: (0, 0))


_tc1 = pl.pallas_call(
    _tc1_body,
    grid=_GRID,
    in_specs=[_hist_spec, _row_spec(F_IN), _full_spec(F_IN, H)],
    out_specs=_row_spec(H),
    out_shape=jax.ShapeDtypeStruct((NPAD, H), jnp.float32),
)

_tc2 = pl.pallas_call(
    _tc2_body,
    grid=_GRID,
    in_specs=[_hist_spec, _part_spec(H), _row_spec(H),
              _full_spec(1, H), _full_spec(H, H)],
    out_specs=_row_spec(H),
    out_shape=jax.ShapeDtypeStruct((NPAD, H), jnp.float32),
)

_tc3 = pl.pallas_call(
    _tc3_body,
    grid=_GRID,
    in_specs=[_hist_spec, _part_spec(H), _row_spec(H), _full_spec(1, H)],
    out_specs=_row_spec(H),
    out_shape=jax.ShapeDtypeStruct((NPAD, H), jnp.float32),
)


def kernel(x, edge_index, W1, b1, W2, b2):
    ei = edge_index.astype(jnp.int32)
    col_t = ei[1].reshape(NW, EPT)
    sent = N + jnp.arange(EPAD - E, dtype=jnp.int32) % (NPAD - N)
    eip = jnp.concatenate(
        [ei, jnp.broadcast_to(sent, (2, EPAD - E))], axis=1)
    rc = eip.reshape(2, NW, NCH, CHUNK).transpose(1, 2, 0, 3).reshape(
        NW * NCH, 2, CHUNK)
    xp = jnp.pad(x, ((0, NPAD - N), (0, 0)))

    w2p = jnp.pad(W2, ((0, 0), (0, H - OUT)))
    b2p = jnp.pad(b2, (0, H - OUT)).reshape(1, H)

    hist = _deg(col_t)
    g1 = _tc1(hist, xp, W1)
    s1 = _scatter_h(g1, rc)
    g2 = _tc2(hist, s1, g1, b1.reshape(1, H), w2p)
    s2 = _scatter_h(g2, rc)
    z = _tc3(hist, s2, g2, b2p)
    return z[:N, :OUT]


# speedup vs baseline: 3.3177x; 1.0189x over previous
import functools

import jax
import jax.numpy as jnp
from jax import lax
from jax.experimental import pallas as pl
from jax.experimental.pallas import tpu as pltpu
from jax.experimental.pallas import tpu_sc as plsc

N = 10000
E = 320000
F_IN = 128
H = 128
OUT = 64

NPAD = 10240
NC = 2
NS = 16
NW = NC * NS
EPT = E // NW
CHUNK = 120
NCH = 84
EPAD = NW * NCH * CHUNK
RPT = NPAD // NS
ZR = 64

_mesh = plsc.VectorSubcoreMesh(core_axis_name="c", subcore_axis_name="s")
_sc_params = pltpu.CompilerParams(needs_layout_passes=False)


def _deg_body(col_hbm, out_hbm, colbuf, hist):
    cid = lax.axis_index("c")
    sid = lax.axis_index("s")
    wid = sid * NC + cid
    pltpu.sync_copy(col_hbm.at[wid], colbuf)

    def zero(j, carry):
        hist[pl.ds(j * 16, 16)] = jnp.zeros((16,), jnp.float32)
        return carry

    lax.fori_loop(0, NPAD // 16, zero, 0)

    ones = jnp.ones((16,), jnp.float32)

    def body(j, carry):
        for u in range(5):
            idx = colbuf[pl.ds((5 * j + u) * 16, 16)]
            plsc.addupdate_scatter(hist, [idx], ones)
        return carry

    lax.fori_loop(0, EPT // 80, body, 0)
    pltpu.sync_copy(hist, out_hbm.at[wid])


_deg = pl.kernel(
    _deg_body,
    mesh=_mesh,
    out_type=jax.ShapeDtypeStruct((NW, NPAD), jnp.float32),
    scratch_types=[
        pltpu.VMEM((EPT,), jnp.int32),
        pltpu.VMEM((NPAD,), jnp.float32),
    ],
    compiler_params=_sc_params,
)


def _scatter_body(feat, g_hbm, rc_hbm, out_hbm,
                  ibuf, gbuf, s_sh,
                  gsem0, gsem1, gsem2, ssem0, ssem1, ssem2,
                  isem0, isem1, isem2, isem3, isem4, isem5):
    cid = lax.axis_index("c")
    sid = lax.axis_index("s")
    wid = sid * NC + cid
    base = wid * NCH
    gsems = (gsem0, gsem1, gsem2)
    ssems = (ssem0, ssem1, ssem2)
    isems = (isem0, isem1, isem2, isem3, isem4, isem5)

    for q in range(4):
        pltpu.async_copy(rc_hbm.at[base + q], ibuf.at[q], isems[q])

    def zrow(r, carry):
        def zcol(c, carry2):
            gbuf[2, r, pl.ds(c * 16, 16)] = jnp.zeros((16,), jnp.float32)
            return carry2

        return lax.fori_loop(0, feat // 16, zcol, carry)

    lax.fori_loop(0, CHUNK, zrow, 0)

    def zfill(r, carry):
        pltpu.sync_copy(gbuf.at[2],
                        s_sh.at[pl.ds(sid * RPT + r * CHUNK, CHUNK)])
        return carry

    lax.fori_loop(0, RPT // CHUNK, zfill, 0)
    if RPT % CHUNK:
        pltpu.sync_copy(
            gbuf.at[2].at[pl.ds(0, RPT % CHUNK)],
            s_sh.at[pl.ds(sid * RPT + (RPT // CHUNK) * CHUNK, RPT % CHUNK)])

    for u in range(2):
        pltpu.make_async_copy(rc_hbm.at[base + u], ibuf.at[u],
                              isems[u]).wait()
        pltpu.async_copy(g_hbm.at[ibuf.at[u, 0]], gbuf.at[u], gsems[u])
    plsc.subcore_barrier()

    def six(p, carry):
        j0 = 6 * p
        for u in range(6):
            j = j0 + u
            b = u % 3
            b2 = (u + 2) % 3
            q2 = (u + 2) % 6
            q4 = (u + 4) % 6
            pltpu.make_async_copy(g_hbm.at[ibuf.at[u, 0]], gbuf.at[b],
                                  gsems[b]).wait()
            pltpu.async_copy(gbuf.at[b], s_sh.at[ibuf.at[u, 1]],
                             ssems[b], add=True)

            @pl.when(j + 2 < NCH)
            def _():
                @pl.when(j >= 1)
                def _():
                    pltpu.make_async_copy(gbuf.at[b2], s_sh.at[ibuf.at[u, 1]],
                                          ssems[b2]).wait()

                pltpu.make_async_copy(rc_hbm.at[base + j + 2], ibuf.at[q2],
                                      isems[q2]).wait()
                pltpu.async_copy(g_hbm.at[ibuf.at[q2, 0]], gbuf.at[b2],
                                 gsems[b2])

            @pl.when(j + 4 < NCH)
            def _():
                pltpu.async_copy(rc_hbm.at[base + j + 4], ibuf.at[q4],
                                 isems[q4])

        return carry

    lax.fori_loop(0, NCH // 6, six, 0)
    for j in (NCH - 3, NCH - 2, NCH - 1):
        b = j % 3
        pltpu.make_async_copy(gbuf.at[b], s_sh.at[ibuf.at[j % 6, 1]],
                              ssems[b]).wait()
    plsc.subcore_barrier()

    def out(r, carry):
        sl = pl.ds(sid * RPT + r * ZR, ZR)
        pltpu.sync_copy(s_sh.at[sl], out_hbm.at[cid, sl])
        return carry

    lax.fori_loop(0, RPT // ZR, out, 0)


def _make_scatter(feat):
    return pl.kernel(
        functools.partial(_scatter_body, feat),
        mesh=_mesh,
        out_type=jax.ShapeDtypeStruct((NC, NPAD, feat), jnp.float32),
        scratch_types=[
            pltpu.VMEM((6, 2, CHUNK), jnp.int32),
            pltpu.VMEM((3, CHUNK, feat), jnp.float32),
            pltpu.VMEM_SHARED((NPAD, feat), jnp.float32),
        ] + [pltpu.SemaphoreType.DMA] * 12,
        compiler_params=_sc_params,
    )


_scatter_h = _make_scatter(H)


NB = 1024


def _dinv_of(hist_ref):
    deg = jnp.sum(hist_ref[...], axis=0) + 1.0
    return lax.rsqrt(deg)[:, None]


def _tc1_body(hist_ref, x_ref, w_ref, g_ref):
    h = jnp.dot(x_ref[...], w_ref[...], preferred_element_type=jnp.float32)
    g_ref[...] = h * _dinv_of(hist_ref)


def _tc2_body(hist_ref, s_ref, g1_ref, b1_ref, w2_ref, g2_ref):
    dinv = _dinv_of(hist_ref)
    s = s_ref[0] + s_ref[1] + g1_ref[...]
    h1 = jnp.maximum(s * dinv + b1_ref[...], 0.0)
    g2_ref[...] = jnp.dot(h1, w2_ref[...],
                          preferred_element_type=jnp.float32) * dinv


def _tc3_body(hist_ref, s_ref, g2_ref, b2_ref, z_ref):
    s = s_ref[0] + s_ref[1] + g2_ref[...]
    z_ref[...] = (s * _dinv_of(hist_ref))[:, :OUT] + b2_ref[...]


_GRID = (NPAD // NB,)
_hist_spec = pl.BlockSpec((NW, NB), lambda i: (0, i))


def _row_spec(f):
    return pl.BlockSpec((NB, f), lambda i: (i, 0))


def _part_spec(f):
    return pl.BlockSpec((NC, NB, f), lambda i: (0, i, 0))


def _full_spec(r, c):
    return pl.BlockSpec((r, c), lambda i: (0, 0))


_tc1 = pl.pallas_call(
    _tc1_body,
    grid=_GRID,
    in_specs=[_hist_spec, _row_spec(F_IN), _full_spec(F_IN, H)],
    out_specs=_row_spec(H),
    out_shape=jax.ShapeDtypeStruct((NPAD, H), jnp.float32),
)

_tc2 = pl.pallas_call(
    _tc2_body,
    grid=_GRID,
    in_specs=[_hist_spec, _part_spec(H), _row_spec(H),
              _full_spec(1, H), _full_spec(H, H)],
    out_specs=_row_spec(H),
    out_shape=jax.ShapeDtypeStruct((NPAD, H), jnp.float32),
)

_tc3 = pl.pallas_call(
    _tc3_body,
    grid=_GRID,
    in_specs=[_hist_spec, _part_spec(H), _row_spec(H), _full_spec(1, OUT)],
    out_specs=_row_spec(OUT),
    out_shape=jax.ShapeDtypeStruct((N, OUT), jnp.float32),
)


def kernel(x, edge_index, W1, b1, W2, b2):
    ei = edge_index.astype(jnp.int32)
    col_t = ei[1].reshape(NW, EPT)
    sent = N + jnp.arange(EPAD - E, dtype=jnp.int32) % (NPAD - N)
    eip = jnp.concatenate(
        [ei, jnp.broadcast_to(sent, (2, EPAD - E))], axis=1)
    rc = eip.reshape(2, NW, NCH, CHUNK).transpose(1, 2, 0, 3).reshape(
        NW * NCH, 2, CHUNK)

    w2p = jnp.pad(W2, ((0, 0), (0, H - OUT)))

    hist = _deg(col_t)
    g1 = _tc1(hist, x, W1)
    s1 = _scatter_h(g1, rc)
    g2 = _tc2(hist, s1, g1, b1.reshape(1, H), w2p)
    s2 = _scatter_h(g2, rc)
    return _tc3(hist, s2, g2, b2.reshape(1, OUT))
